# Initial kernel scaffold; baseline (speedup 1.0000x reference)
import jax
import jax.numpy as jnp
from jax.experimental import pallas as pl


def kernel(x, edge_index, edge_attr, W1, b1, Wm1, bm1, Wm2, bm2, Wm3, bm3, Wa, ba):
    raise NotImplementedError("write your pallas kernel here")


# speedup vs baseline: 2.6785x; 2.6785x over previous
import functools

import jax
import jax.numpy as jnp
from jax import lax
from jax.experimental import pallas as pl
from jax.experimental.pallas import tpu as pltpu
from jax.experimental.pallas import tpu_sc as plsc

N_NODES = 10000
N_EDGES = 640000
D = 128
EDGE_DIM = 16
NC, NS = 2, 16
NW = NC * NS
EPW = N_EDGES // NW
CHUNK = 80
NCHUNKS = EPW // CHUNK
N_PAD = 10240
NPS = N_PAD // NS

@functools.lru_cache(maxsize=None)
def _get_mesh():
    return plsc.VectorSubcoreMesh(core_axis_name="c", subcore_axis_name="s",
                                  num_cores=NC, num_subcores=NS)


def _sc_pass_body(write_h, h_hbm, src_hbm, dst_hbm, m_hbm, zeros_hbm,
                  *refs):
    if write_h:
        hnew_hbm, agg_hbm = refs[0], refs[1]
        scratch = refs[2:]
    else:
        agg_hbm = refs[0]
        scratch = refs[1:]
    (src_v, dst_v, hrow_v, mrow_v, agg_sh, sem_i, sem_h, sem_g) = scratch

    c = lax.axis_index("c")
    s = lax.axis_index("s")
    wid = c * NS + s

    pltpu.sync_copy(zeros_hbm.at[pl.ds(s * NPS, NPS)],
                    agg_sh.at[pl.ds(s * NPS, NPS)])
    plsc.subcore_barrier()

    base0 = wid * EPW

    def chunk_body(i, carry):
        base = base0 + i * CHUNK
        cp_src = pltpu.async_copy(src_hbm.at[pl.ds(base, CHUNK)], src_v, sem_i)
        cp_dst = pltpu.async_copy(dst_hbm.at[pl.ds(base, CHUNK)], dst_v, sem_i)
        cp_h = pltpu.async_copy(h_hbm.at[pl.ds(base, CHUNK)], hrow_v, sem_h)
        cp_src.wait()
        cp_g = pltpu.async_copy(m_hbm.at[src_v], mrow_v, sem_g)
        cp_h.wait()
        cp_g.wait()

        def row_body(r, carry2):
            for j in range(D // 16):
                a = hrow_v[r, pl.ds(j * 16, 16)]
                b = mrow_v[r, pl.ds(j * 16, 16)]
                hrow_v[r, pl.ds(j * 16, 16)] = jnp.maximum(a + b, 0.0)
            return carry2

        lax.fori_loop(0, CHUNK, row_body, 0, unroll=2)
        cp_dst.wait()
        pltpu.sync_copy(hrow_v, agg_sh.at[dst_v], add=True)
        if write_h:
            pltpu.sync_copy(hrow_v, hnew_hbm.at[pl.ds(base, CHUNK)])
        return carry

    lax.fori_loop(0, NCHUNKS, chunk_body, 0)
    plsc.subcore_barrier()
    pltpu.sync_copy(agg_sh.at[pl.ds(s * NPS, NPS)],
                    agg_hbm.at[c, pl.ds(s * NPS, NPS)])


@functools.lru_cache(maxsize=None)
def _make_sc_pass(write_h):
    out_type = [jax.ShapeDtypeStruct((NC, N_PAD, D), jnp.float32)]
    if write_h:
        out_type = [jax.ShapeDtypeStruct((N_EDGES, D), jnp.float32)] + out_type
    return pl.kernel(
        functools.partial(_sc_pass_body, write_h),
        mesh=_get_mesh(),
        out_type=out_type,
        scratch_types=[
            pltpu.VMEM((CHUNK,), jnp.int32),
            pltpu.VMEM((CHUNK,), jnp.int32),
            pltpu.VMEM((CHUNK, D), jnp.float32),
            pltpu.VMEM((CHUNK, D), jnp.float32),
            pltpu.VMEM_SHARED((N_PAD, D), jnp.float32),
            pltpu.SemaphoreType.DMA,
            pltpu.SemaphoreType.DMA,
            pltpu.SemaphoreType.DMA,
        ],
    )


def _tc_matmul(a, w, b, relu=False, block_m=2000):
    m, k = a.shape
    h = w.shape[1]
    b2 = b.reshape(1, h)

    def body(a_ref, w_ref, b_ref, o_ref):
        acc = jnp.dot(a_ref[...], w_ref[...],
                      preferred_element_type=jnp.float32) + b_ref[...]
        o_ref[...] = jnp.maximum(acc, 0.0) if relu else acc

    return pl.pallas_call(
        body,
        grid=(m // block_m,),
        in_specs=[
            pl.BlockSpec((block_m, k), lambda i: (i, 0)),
            pl.BlockSpec((k, h), lambda i: (0, 0)),
            pl.BlockSpec((1, h), lambda i: (0, 0)),
        ],
        out_specs=pl.BlockSpec((block_m, h), lambda i: (i, 0)),
        out_shape=jax.ShapeDtypeStruct((m, h), jnp.float32),
    )(a, w, b2)


def _tc_msg(agg, w, b):
    b2 = b.reshape(1, D)

    def body(a_ref, w_ref, b_ref, o_ref):
        s = a_ref[0] + a_ref[1]
        o_ref[...] = jnp.dot(s, w_ref[...],
                             preferred_element_type=jnp.float32) + b_ref[...]

    bm = 2000
    return pl.pallas_call(
        body,
        grid=(N_NODES // bm,),
        in_specs=[
            pl.BlockSpec((NC, bm, D), lambda i: (0, i, 0)),
            pl.BlockSpec((D, D), lambda i: (0, 0)),
            pl.BlockSpec((1, D), lambda i: (0, 0)),
        ],
        out_specs=pl.BlockSpec((bm, D), lambda i: (i, 0)),
        out_shape=jax.ShapeDtypeStruct((N_NODES, D), jnp.float32),
    )(agg, w, b2)


def _tc_final(x, agg, wa_x, wa_h, ba):
    b2 = ba.reshape(1, D)

    def body(x_ref, a_ref, wx_ref, wh_ref, b_ref, o_ref):
        s = a_ref[0] + a_ref[1]
        acc = (jnp.dot(x_ref[...], wx_ref[...], preferred_element_type=jnp.float32)
               + jnp.dot(s, wh_ref[...], preferred_element_type=jnp.float32)
               + b_ref[...])
        o_ref[...] = jnp.maximum(acc, 0.0)

    bm = 2000
    return pl.pallas_call(
        body,
        grid=(N_NODES // bm,),
        in_specs=[
            pl.BlockSpec((bm, x.shape[1]), lambda i: (i, 0)),
            pl.BlockSpec((NC, bm, D), lambda i: (0, i, 0)),
            pl.BlockSpec((x.shape[1], D), lambda i: (0, 0)),
            pl.BlockSpec((D, D), lambda i: (0, 0)),
            pl.BlockSpec((1, D), lambda i: (0, 0)),
        ],
        out_specs=pl.BlockSpec((bm, D), lambda i: (i, 0)),
        out_shape=jax.ShapeDtypeStruct((N_NODES, D), jnp.float32),
    )(x, agg, wa_x, wa_h, b2)


def kernel(x, edge_index, edge_attr, W1, b1, Wm1, bm1, Wm2, bm2, Wm3, bm3, Wa, ba):
    src = edge_index[0].astype(jnp.int32)
    dst = edge_index[1].astype(jnp.int32)
    nd = x.shape[1]

    xw = _tc_matmul(x, W1[:nd], jnp.zeros((D,), jnp.float32))
    eproj = _tc_matmul(edge_attr, W1[nd:], b1, block_m=4000)
    zeros = jnp.zeros((N_PAD, D), jnp.float32)

    sc_pass_wh = _make_sc_pass(True)
    sc_pass_agg = _make_sc_pass(False)
    h0, agg = sc_pass_wh(eproj, src, dst, xw, zeros)
    m1 = _tc_msg(agg, Wm1, bm1)
    h1, agg = sc_pass_wh(h0, src, dst, m1, zeros)
    m2 = _tc_msg(agg, Wm2, bm2)
    h2, agg = sc_pass_wh(h1, src, dst, m2, zeros)
    m3 = _tc_msg(agg, Wm3, bm3)
    (agg4,) = sc_pass_agg(h2, src, dst, m3, zeros)
    return _tc_final(x, agg4, Wa[:nd], Wa[nd:], ba)


# speedup vs baseline: 3.4865x; 1.3016x over previous
import functools

import jax
import jax.numpy as jnp
from jax import lax
from jax.experimental import pallas as pl
from jax.experimental.pallas import tpu as pltpu
from jax.experimental.pallas import tpu_sc as plsc

N_NODES = 10000
N_EDGES = 640000
D = 128
EDGE_DIM = 16
NC, NS = 2, 16
NW = NC * NS
EPW = N_EDGES // NW
CHUNK = 80
NCHUNKS = EPW // CHUNK
N_PAD = 10240
NPS = N_PAD // NS

@functools.lru_cache(maxsize=None)
def _get_mesh():
    return plsc.VectorSubcoreMesh(core_axis_name="c", subcore_axis_name="s",
                                  num_cores=NC, num_subcores=NS)


def _sc_pass_body(write_h, h_hbm, src_hbm, dst_hbm, m_hbm, zeros_hbm,
                  *refs):
    if write_h:
        hnew_hbm, agg_hbm = refs[0], refs[1]
        scratch = refs[2:]
    else:
        agg_hbm = refs[0]
        scratch = refs[1:]
    (src0, src1, dst0, dst1, h0, h1, m0, m1, agg_sh, *sems) = scratch
    src_v, dst_v, hv, mv = [src0, src1], [dst0, dst1], [h0, h1], [m0, m1]
    sem_src = sems[0:2]
    sem_dst = sems[2:4]
    sem_h = sems[4:6]
    sem_g = sems[6:8]
    sem_sc = sems[8:10]
    sem_st = sems[10:12]

    c = lax.axis_index("c")
    s = lax.axis_index("s")
    wid = c * NS + s

    pltpu.sync_copy(zeros_hbm.at[pl.ds(s * NPS, NPS)],
                    agg_sh.at[pl.ds(s * NPS, NPS)])
    plsc.subcore_barrier()

    base0 = wid * EPW

    def issue_loads(i, b):
        base = base0 + i * CHUNK
        pltpu.async_copy(src_hbm.at[pl.ds(base, CHUNK)], src_v[b], sem_src[b])
        pltpu.async_copy(dst_hbm.at[pl.ds(base, CHUNK)], dst_v[b], sem_dst[b])
        pltpu.async_copy(h_hbm.at[pl.ds(base, CHUNK)], hv[b], sem_h[b])

    def wait(src, dst, sem):
        pltpu.make_async_copy(src, dst, sem).wait()

    def issue_gather(b):
        pltpu.async_copy(m_hbm.at[src_v[b]], mv[b], sem_g[b])

    def wait_free(b):
        wait(hv[b], agg_sh.at[dst_v[b]], sem_sc[b])
        if write_h:
            wait(hv[b], hnew_hbm.at[pl.ds(0, CHUNK)], sem_st[b])

    issue_loads(0, 0)
    wait(src_hbm.at[pl.ds(0, CHUNK)], src_v[0], sem_src[0])
    issue_gather(0)

    def step(i, b, free_pred, prefetch_pred):
        nb = 1 - b

        if free_pred is None:
            wait_free(nb)
        else:
            @pl.when(free_pred)
            def _():
                wait_free(nb)

        def _prefetch():
            issue_loads(i + 1, nb)
            wait(src_hbm.at[pl.ds(0, CHUNK)], src_v[nb], sem_src[nb])
            issue_gather(nb)

        if prefetch_pred is None:
            _prefetch()
        else:
            @pl.when(prefetch_pred)
            def _():
                _prefetch()

        wait(h_hbm.at[pl.ds(0, CHUNK)], hv[b], sem_h[b])
        wait(m_hbm.at[src_v[b]], mv[b], sem_g[b])
        wait(dst_hbm.at[pl.ds(0, CHUNK)], dst_v[b], sem_dst[b])

        def row_body(r, carry2):
            for j in range(D // 16):
                sl = pl.ds(j * 16, 16)
                hv[b][r, sl] = jnp.maximum(hv[b][r, sl] + mv[b][r, sl], 0.0)
            return carry2

        lax.fori_loop(0, CHUNK, row_body, 0, unroll=4)

        pltpu.async_copy(hv[b], agg_sh.at[dst_v[b]], sem_sc[b], add=True)
        if write_h:
            pltpu.async_copy(hv[b], hnew_hbm.at[pl.ds(base0 + i * CHUNK, CHUNK)],
                             sem_st[b])

    def outer(ii, carry):
        step(ii * 2, 0, free_pred=ii >= 1, prefetch_pred=None)
        step(ii * 2 + 1, 1, free_pred=None,
             prefetch_pred=ii < NCHUNKS // 2 - 1)
        return carry

    lax.fori_loop(0, NCHUNKS // 2, outer, 0)
    wait_free((NCHUNKS - 1) % 2)
    plsc.subcore_barrier()
    pltpu.sync_copy(agg_sh.at[pl.ds(s * NPS, NPS)],
                    agg_hbm.at[c, pl.ds(s * NPS, NPS)])


@functools.lru_cache(maxsize=None)
def _make_sc_pass(write_h):
    out_type = [jax.ShapeDtypeStruct((NC, N_PAD, D), jnp.float32)]
    if write_h:
        out_type = [jax.ShapeDtypeStruct((N_EDGES, D), jnp.float32)] + out_type
    return pl.kernel(
        functools.partial(_sc_pass_body, write_h),
        mesh=_get_mesh(),
        out_type=out_type,
        scratch_types=(
            [pltpu.VMEM((CHUNK,), jnp.int32)] * 4
            + [pltpu.VMEM((CHUNK, D), jnp.float32)] * 4
            + [pltpu.VMEM_SHARED((N_PAD, D), jnp.float32)]
            + [pltpu.SemaphoreType.DMA] * 12
        ),
    )


def _tc_matmul(a, w, b, relu=False, block_m=2000):
    m, k = a.shape
    h = w.shape[1]
    b2 = b.reshape(1, h)

    def body(a_ref, w_ref, b_ref, o_ref):
        acc = jnp.dot(a_ref[...], w_ref[...],
                      preferred_element_type=jnp.float32) + b_ref[...]
        o_ref[...] = jnp.maximum(acc, 0.0) if relu else acc

    return pl.pallas_call(
        body,
        grid=(m // block_m,),
        in_specs=[
            pl.BlockSpec((block_m, k), lambda i: (i, 0)),
            pl.BlockSpec((k, h), lambda i: (0, 0)),
            pl.BlockSpec((1, h), lambda i: (0, 0)),
        ],
        out_specs=pl.BlockSpec((block_m, h), lambda i: (i, 0)),
        out_shape=jax.ShapeDtypeStruct((m, h), jnp.float32),
    )(a, w, b2)


def _tc_msg(agg, w, b):
    b2 = b.reshape(1, D)

    def body(a_ref, w_ref, b_ref, o_ref):
        s = a_ref[0] + a_ref[1]
        o_ref[...] = jnp.dot(s, w_ref[...],
                             preferred_element_type=jnp.float32) + b_ref[...]

    bm = 2000
    return pl.pallas_call(
        body,
        grid=(N_NODES // bm,),
        in_specs=[
            pl.BlockSpec((NC, bm, D), lambda i: (0, i, 0)),
            pl.BlockSpec((D, D), lambda i: (0, 0)),
            pl.BlockSpec((1, D), lambda i: (0, 0)),
        ],
        out_specs=pl.BlockSpec((bm, D), lambda i: (i, 0)),
        out_shape=jax.ShapeDtypeStruct((N_NODES, D), jnp.float32),
    )(agg, w, b2)


def _tc_final(x, agg, wa_x, wa_h, ba):
    b2 = ba.reshape(1, D)

    def body(x_ref, a_ref, wx_ref, wh_ref, b_ref, o_ref):
        s = a_ref[0] + a_ref[1]
        acc = (jnp.dot(x_ref[...], wx_ref[...], preferred_element_type=jnp.float32)
               + jnp.dot(s, wh_ref[...], preferred_element_type=jnp.float32)
               + b_ref[...])
        o_ref[...] = jnp.maximum(acc, 0.0)

    bm = 2000
    return pl.pallas_call(
        body,
        grid=(N_NODES // bm,),
        in_specs=[
            pl.BlockSpec((bm, x.shape[1]), lambda i: (i, 0)),
            pl.BlockSpec((NC, bm, D), lambda i: (0, i, 0)),
            pl.BlockSpec((x.shape[1], D), lambda i: (0, 0)),
            pl.BlockSpec((D, D), lambda i: (0, 0)),
            pl.BlockSpec((1, D), lambda i: (0, 0)),
        ],
        out_specs=pl.BlockSpec((bm, D), lambda i: (i, 0)),
        out_shape=jax.ShapeDtypeStruct((N_NODES, D), jnp.float32),
    )(x, agg, wa_x, wa_h, b2)


def kernel(x, edge_index, edge_attr, W1, b1, Wm1, bm1, Wm2, bm2, Wm3, bm3, Wa, ba):
    src = edge_index[0].astype(jnp.int32)
    dst = edge_index[1].astype(jnp.int32)
    nd = x.shape[1]

    xw = _tc_matmul(x, W1[:nd], jnp.zeros((D,), jnp.float32))
    eproj = _tc_matmul(edge_attr, W1[nd:], b1, block_m=4000)
    zeros = jnp.zeros((N_PAD, D), jnp.float32)

    sc_pass_wh = _make_sc_pass(True)
    sc_pass_agg = _make_sc_pass(False)
    h0, agg = sc_pass_wh(eproj, src, dst, xw, zeros)
    m1 = _tc_msg(agg, Wm1, bm1)
    h1, agg = sc_pass_wh(h0, src, dst, m1, zeros)
    m2 = _tc_msg(agg, Wm2, bm2)
    h2, agg = sc_pass_wh(h1, src, dst, m2, zeros)
    m3 = _tc_msg(agg, Wm3, bm3)
    (agg4,) = sc_pass_agg(h2, src, dst, m3, zeros)
    return _tc_final(x, agg4, Wa[:nd], Wa[nd:], ba)


# speedup vs baseline: 4.3225x; 1.2398x over previous
import functools

import jax
import jax.numpy as jnp
from jax import lax
from jax.experimental import pallas as pl
from jax.experimental.pallas import tpu as pltpu
from jax.experimental.pallas import tpu_sc as plsc

N_NODES = 10000
N_EDGES = 640000
D = 128
EDGE_DIM = 16
NC, NS = 2, 16
NW = NC * NS
N_PAD = 10240
NPS = N_PAD // NS
EPW = N_EDGES // NW
CHUNK = 40
NCHUNKS = EPW // CHUNK
NBUF = 4
PD_LOAD = 2
PD_G = 1
OUTER = NCHUNKS // NBUF


@functools.lru_cache(maxsize=None)
def _get_mesh():
    return plsc.VectorSubcoreMesh(core_axis_name="c", subcore_axis_name="s",
                                  num_cores=NC, num_subcores=NS)


def _sc_pass_body(write_h, h_hbm, src_hbm, dst_hbm, m_hbm, zeros_hbm, *refs):
    if write_h:
        hnew_hbm, agg_hbm = refs[0], refs[1]
        scratch = refs[2:]
    else:
        agg_hbm = refs[0]
        scratch = refs[1:]
    src_v = list(scratch[0:NBUF])
    dst_v = list(scratch[NBUF:2 * NBUF])
    hv = list(scratch[2 * NBUF:3 * NBUF])
    mv = list(scratch[3 * NBUF:4 * NBUF])
    agg_sh = scratch[4 * NBUF]
    sems = scratch[4 * NBUF + 1:]
    sem_src = sems[0:NBUF]
    sem_dst = sems[NBUF:2 * NBUF]
    sem_h = sems[2 * NBUF:3 * NBUF]
    sem_g = sems[3 * NBUF:4 * NBUF]
    sem_sc = sems[4 * NBUF:5 * NBUF]
    sem_st = sems[5 * NBUF:6 * NBUF]

    c = lax.axis_index("c")
    s = lax.axis_index("s")
    wid = c * NS + s

    pltpu.sync_copy(zeros_hbm.at[pl.ds(s * NPS, NPS)],
                    agg_sh.at[pl.ds(s * NPS, NPS)])
    plsc.subcore_barrier()

    base0 = wid * EPW

    def issue_loads(i, b):
        base = base0 + i * CHUNK
        pltpu.async_copy(src_hbm.at[pl.ds(base, CHUNK)], src_v[b], sem_src[b])
        pltpu.async_copy(dst_hbm.at[pl.ds(base, CHUNK)], dst_v[b], sem_dst[b])
        pltpu.async_copy(h_hbm.at[pl.ds(base, CHUNK)], hv[b], sem_h[b])

    def wait(src, dst, sem):
        pltpu.make_async_copy(src, dst, sem).wait()

    def issue_gather(b):
        pltpu.async_copy(m_hbm.at[src_v[b]], mv[b], sem_g[b])

    def wait_free(b):
        wait(hv[b], agg_sh.at[dst_v[b]], sem_sc[b])
        if write_h:
            wait(hv[b], hnew_hbm.at[pl.ds(0, CHUNK)], sem_st[b])

    def _when(pred, fn):
        if pred is None:
            fn()
        else:
            pl.when(pred)(fn)

    def i_max(b):
        return (OUTER - 1) * NBUF + b

    for i0 in range(PD_LOAD):
        issue_loads(i0, i0)
    for i0 in range(PD_G):
        wait(src_hbm.at[pl.ds(0, CHUNK)], src_v[i0], sem_src[i0])
        issue_gather(i0)

    def step(ii, b):
        i = ii * NBUF + b
        b_load = (b + PD_LOAD) % NBUF
        b_g = (b + PD_G) % NBUF

        _when(None if b >= NBUF - PD_LOAD else ii >= 1,
              lambda: wait_free(b_load))
        _when(None if i_max(b) + PD_LOAD <= NCHUNKS - 1 else ii < OUTER - 1,
              lambda: issue_loads(i + PD_LOAD, b_load))

        def _fire_gather():
            wait(src_hbm.at[pl.ds(0, CHUNK)], src_v[b_g], sem_src[b_g])
            issue_gather(b_g)

        _when(None if i_max(b) + PD_G <= NCHUNKS - 1 else ii < OUTER - 1,
              _fire_gather)

        wait(h_hbm.at[pl.ds(0, CHUNK)], hv[b], sem_h[b])
        wait(m_hbm.at[src_v[b]], mv[b], sem_g[b])
        wait(dst_hbm.at[pl.ds(0, CHUNK)], dst_v[b], sem_dst[b])

        def row_body(r, carry2):
            for j in range(D // 16):
                sl = pl.ds(j * 16, 16)
                hv[b][r, sl] = jnp.maximum(hv[b][r, sl] + mv[b][r, sl], 0.0)
            return carry2

        lax.fori_loop(0, CHUNK, row_body, 0, unroll=4)

        pltpu.async_copy(hv[b], agg_sh.at[dst_v[b]], sem_sc[b], add=True)
        if write_h:
            pltpu.async_copy(hv[b],
                             hnew_hbm.at[pl.ds(base0 + i * CHUNK, CHUNK)],
                             sem_st[b])

    def outer_body(ii, carry):
        for b in range(NBUF):
            step(ii, b)
        return carry

    lax.fori_loop(0, OUTER, outer_body, 0)
    for i0 in range(NCHUNKS - (NBUF - PD_LOAD), NCHUNKS):
        wait_free(i0 % NBUF)
    plsc.subcore_barrier()
    pltpu.sync_copy(agg_sh.at[pl.ds(s * NPS, NPS)],
                    agg_hbm.at[c, pl.ds(s * NPS, NPS)])


@functools.lru_cache(maxsize=None)
def _make_sc_pass(write_h):
    out_type = [jax.ShapeDtypeStruct((NC, N_PAD, D), jnp.float32)]
    if write_h:
        out_type = [jax.ShapeDtypeStruct((N_EDGES, D), jnp.float32)] + out_type
    return pl.kernel(
        functools.partial(_sc_pass_body, write_h),
        mesh=_get_mesh(),
        out_type=out_type,
        scratch_types=(
            [pltpu.VMEM((CHUNK,), jnp.int32)] * (2 * NBUF)
            + [pltpu.VMEM((CHUNK, D), jnp.float32)] * (2 * NBUF)
            + [pltpu.VMEM_SHARED((N_PAD, D), jnp.float32)]
            + [pltpu.SemaphoreType.DMA] * (6 * NBUF)
        ),
    )


def _tc_matmul(a, w, b, relu=False, block_m=2000):
    m, k = a.shape
    h = w.shape[1]
    b2 = b.reshape(1, h)

    def body(a_ref, w_ref, b_ref, o_ref):
        acc = jnp.dot(a_ref[...], w_ref[...],
                      preferred_element_type=jnp.float32) + b_ref[...]
        o_ref[...] = jnp.maximum(acc, 0.0) if relu else acc

    return pl.pallas_call(
        body,
        grid=(m // block_m,),
        in_specs=[
            pl.BlockSpec((block_m, k), lambda i: (i, 0)),
            pl.BlockSpec((k, h), lambda i: (0, 0)),
            pl.BlockSpec((1, h), lambda i: (0, 0)),
        ],
        out_specs=pl.BlockSpec((block_m, h), lambda i: (i, 0)),
        out_shape=jax.ShapeDtypeStruct((m, h), jnp.float32),
    )(a, w, b2)


def _tc_msg(agg, w, b):
    b2 = b.reshape(1, D)
    bm = 2048

    def body(a_ref, w_ref, b_ref, o_ref):
        s = a_ref[0] + a_ref[1]
        o_ref[...] = jnp.dot(s, w_ref[...],
                             preferred_element_type=jnp.float32) + b_ref[...]

    return pl.pallas_call(
        body,
        grid=(N_PAD // bm,),
        in_specs=[
            pl.BlockSpec((NC, bm, D), lambda i: (0, i, 0)),
            pl.BlockSpec((D, D), lambda i: (0, 0)),
            pl.BlockSpec((1, D), lambda i: (0, 0)),
        ],
        out_specs=pl.BlockSpec((bm, D), lambda i: (i, 0)),
        out_shape=jax.ShapeDtypeStruct((N_PAD, D), jnp.float32),
    )(agg, w, b2)


def _tc_final(x, agg, wa_x, wa_h, ba):
    b2 = ba.reshape(1, D)
    bm = 2000

    def body(x_ref, a_ref, wx_ref, wh_ref, b_ref, o_ref):
        s = a_ref[0] + a_ref[1]
        acc = (jnp.dot(x_ref[...], wx_ref[...], preferred_element_type=jnp.float32)
               + jnp.dot(s, wh_ref[...], preferred_element_type=jnp.float32)
               + b_ref[...])
        o_ref[...] = jnp.maximum(acc, 0.0)

    return pl.pallas_call(
        body,
        grid=(N_NODES // bm,),
        in_specs=[
            pl.BlockSpec((bm, x.shape[1]), lambda i: (i, 0)),
            pl.BlockSpec((NC, bm, D), lambda i: (0, i, 0)),
            pl.BlockSpec((x.shape[1], D), lambda i: (0, 0)),
            pl.BlockSpec((D, D), lambda i: (0, 0)),
            pl.BlockSpec((1, D), lambda i: (0, 0)),
        ],
        out_specs=pl.BlockSpec((bm, D), lambda i: (i, 0)),
        out_shape=jax.ShapeDtypeStruct((N_NODES, D), jnp.float32),
    )(x, agg, wa_x, wa_h, b2)


def kernel(x, edge_index, edge_attr, W1, b1, Wm1, bm1, Wm2, bm2, Wm3, bm3, Wa, ba):
    src = edge_index[0].astype(jnp.int32)
    dst = edge_index[1].astype(jnp.int32)
    nd = x.shape[1]

    xw_full = _tc_matmul(x, W1[:nd], jnp.zeros((D,), jnp.float32))
    xw = jnp.zeros((N_PAD, D), jnp.float32).at[:N_NODES].set(xw_full)
    ep = _tc_matmul(edge_attr, W1[nd:], b1, block_m=4000)
    zeros = jnp.zeros((N_PAD, D), jnp.float32)

    sc_pass_wh = _make_sc_pass(True)
    sc_pass_agg = _make_sc_pass(False)
    h0, agg = sc_pass_wh(ep, src, dst, xw, zeros)
    m1 = _tc_msg(agg, Wm1, bm1)
    h1, agg = sc_pass_wh(h0, src, dst, m1, zeros)
    m2 = _tc_msg(agg, Wm2, bm2)
    h2, agg = sc_pass_wh(h1, src, dst, m2, zeros)
    m3 = _tc_msg(agg, Wm3, bm3)
    (agg4,) = sc_pass_agg(h2, src, dst, m3, zeros)
    return _tc_final(x, agg4, Wa[:nd], Wa[nd:], ba)
